# baseline (speedup 1.0000x reference)
import functools

import jax
import jax.numpy as jnp
import numpy as np
from jax import lax
from jax.experimental import pallas as pl
from jax.experimental.pallas import tpu as pltpu
from jax.experimental.pallas import tpu_sc as plsc

N = 10000
FIN = 128
FQK = 128
FV = 128
H = 8
FH = 16
E = 320000
SCALING = float(FH) ** -0.5

NC = 2
NS = 16
HC = H // NC
QW = HC * FH
VW = HC * FH
KVW = QW + VW
EPT = E // NS
CH = 80
NCHUNK = EPT // CH
RPT = 624
RTL = N - NS * RPT
ZCH = 48
ACC_W = 72
DCOL = VW

RB = 400
GRID = N // RB


def _proj_body(x_ref, wt_ref, q_ref, kv_ref):
    blk = jnp.dot(x_ref[...], wt_ref[...], preferred_element_type=jnp.float32)
    q_ref[0] = blk[:, :QW] * SCALING
    q_ref[1] = blk[:, QW:FQK] * SCALING
    kv_ref[0] = jnp.concatenate(
        [blk[:, FQK:FQK + QW], blk[:, 2 * FQK:2 * FQK + VW]], axis=1)
    kv_ref[1] = jnp.concatenate(
        [blk[:, FQK + QW:2 * FQK], blk[:, 2 * FQK + VW:]], axis=1)


def _project(x, wt):
    return pl.pallas_call(
        _proj_body,
        grid=(GRID,),
        in_specs=[
            pl.BlockSpec((RB, FIN), lambda i: (i, 0)),
            pl.BlockSpec((FIN, 2 * FQK + FV), lambda i: (0, 0)),
        ],
        out_specs=[
            pl.BlockSpec((NC, RB, QW), lambda i: (0, i, 0)),
            pl.BlockSpec((NC, RB, KVW), lambda i: (0, i, 0)),
        ],
        out_shape=[
            jax.ShapeDtypeStruct((NC, N, QW), jnp.float32),
            jax.ShapeDtypeStruct((NC, N, KVW), jnp.float32),
        ],
    )(x, wt)


def _sc_edges(src, dst, q2, kv2):
    mesh = plsc.VectorSubcoreMesh(core_axis_name="c", subcore_axis_name="s")

    @functools.partial(
        pl.kernel,
        mesh=mesh,
        compiler_params=pltpu.CompilerParams(
            needs_layout_passes=False, use_tc_tiling_on_sc=False),
        out_type=jax.ShapeDtypeStruct((NC * N, ACC_W), jnp.float32),
        scratch_types=[
            pltpu.VMEM((EPT,), jnp.int32),
            pltpu.VMEM((EPT,), jnp.int32),
            pltpu.VMEM((CH, QW), jnp.float32),
            pltpu.VMEM((CH, QW), jnp.float32),
            pltpu.VMEM((CH, KVW), jnp.float32),
            pltpu.VMEM((CH, KVW), jnp.float32),
            pltpu.VMEM((CH, ACC_W), jnp.float32),
            pltpu.VMEM((CH, ACC_W), jnp.float32),
            pltpu.VMEM((CH,), jnp.int32),
            pltpu.VMEM((CH,), jnp.int32),
            pltpu.VMEM_SHARED((N, ACC_W), jnp.float32),
            [pltpu.SemaphoreType.DMA] * 6,
        ],
    )
    def _attn(src_hbm, dst_hbm, q_hbm, kv_hbm, oacc_hbm,
              sadj, dadj, qr0, qr1, kvr0, kvr1, cb0, cb1,
              sc0, sc1, acc_sh, sems):
        c = lax.axis_index("c")
        s = lax.axis_index("s")
        coff = c * N
        ebase = s * EPT
        qr = (qr0, qr1)
        kvr = (kvr0, kvr1)
        cb = (cb0, cb1)
        sc = (sc0, sc1)
        semq = (sems[0], sems[1])
        semk = (sems[2], sems[3])
        sema = (sems[4], sems[5])

        zeros16 = jnp.zeros((FH,), jnp.float32)
        iota16 = lax.iota(jnp.int32, FH)
        cw = jnp.full((FH,), ACC_W, jnp.int32)

        cp0 = pltpu.async_copy(src_hbm.at[pl.ds(ebase, EPT)], sadj, semq[0])
        cp1 = pltpu.async_copy(dst_hbm.at[pl.ds(ebase, EPT)], dadj, semk[0])
        cp0.wait()
        cp1.wait()

        def _adj(j, _):
            sl = pl.ds(j * FH, FH)
            sadj[sl] = sadj[sl] + coff
            dadj[sl] = dadj[sl] + coff
            return 0

        lax.fori_loop(0, EPT // FH, _adj, 0)

        def _zero_flat(t, _):
            base = t * FH + iota16
            rows = lax.div(base, cw)
            cols = lax.rem(base, cw)
            plsc.store_scatter(cb0, [rows, cols], zeros16)
            plsc.store_scatter(cb1, [rows, cols], zeros16)
            return 0

        lax.fori_loop(0, CH * ACC_W // FH, _zero_flat, 0)
        r0 = s * RPT
        for z in range(RPT // ZCH):
            pltpu.sync_copy(cb0.at[pl.ds(0, ZCH)],
                            acc_sh.at[pl.ds(r0 + z * ZCH, ZCH)])

        @pl.when(s == NS - 1)
        def _zero_tail():
            pltpu.sync_copy(cb0.at[pl.ds(0, RTL)],
                            acc_sh.at[pl.ds(NS * RPT, RTL)])

        plsc.subcore_barrier()

        def _issue_gathers(off, b):
            pltpu.async_copy(q_hbm.at[sadj.at[pl.ds(off, CH)]], qr[b], semq[b])
            pltpu.async_copy(kv_hbm.at[dadj.at[pl.ds(off, CH)]], kvr[b], semk[b])

        idx_e = (2 * iota16) & 15
        idx_o = idx_e | 1
        mask_lo = iota16 < 8

        def _perm(a, idx):
            return jnp.take_along_axis(a, idx, axis=0,
                                       mode="promise_in_bounds")

        def _hadd(a, b):
            sa = _perm(a, idx_e) + _perm(a, idx_o)
            sb = _perm(b, idx_e) + _perm(b, idx_o)
            return jnp.where(mask_lo, sa, sb)

        def _compute(ngroup, qrb, kvrb, cbb):
            @plsc.parallel_loop(0, ngroup, 1, unroll=2)
            def _group(g):
                rows = g * FH + iota16
                for h in range(HC):
                    lvl = []
                    for j in range(FH):
                        e = g * FH + j
                        qv = qrb[e, pl.ds(h * FH, FH)]
                        kv = kvrb[e, pl.ds(h * FH, FH)]
                        lvl.append(qv * kv)
                    while len(lvl) > 1:
                        lvl = [_hadd(lvl[i], lvl[i + 1])
                               for i in range(0, len(lvl), 2)]
                    p = jnp.exp(lvl[0])
                    plsc.store_scatter(
                        cbb, [rows, jnp.full((FH,), DCOL + h, jnp.int32)], p)
                    for j in range(FH):
                        e = g * FH + j
                        pj = _perm(p, jnp.full((FH,), j, jnp.int32))
                        vv = kvrb[e, pl.ds(QW + h * FH, FH)]
                        cbb[e, pl.ds(h * FH, FH)] = pj * vv

        _issue_gathers(0, 0)
        _issue_gathers(CH, 1)

        def _pair(i2, _):
            for b in range(2):
                off = (i2 * 2 + b) * CH
                pltpu.make_async_copy(
                    q_hbm.at[sadj.at[pl.ds(off, CH)]], qr[b], semq[b]).wait()
                pltpu.make_async_copy(
                    kv_hbm.at[dadj.at[pl.ds(off, CH)]], kvr[b], semk[b]).wait()

                @pl.when(i2 >= 1)
                def _wait_scatter():
                    pltpu.make_async_copy(
                        cb[b], acc_sh.at[sc[b]], sema[b]).wait()

                for j in range(CH // FH):
                    sc[b][pl.ds(j * FH, FH)] = (
                        sadj[pl.ds(off + j * FH, FH)] - coff)
                _compute(CH // FH, qr[b], kvr[b], cb[b])
                pltpu.async_copy(cb[b], acc_sh.at[sc[b]], sema[b], add=True)

                @pl.when(i2 < NCHUNK // 2 - 1)
                def _issue_next():
                    _issue_gathers(off + 2 * CH, b)
            return 0

        lax.fori_loop(0, NCHUNK // 2, _pair, 0)
        for b in range(2):
            pltpu.make_async_copy(cb[b], acc_sh.at[sc[b]], sema[b]).wait()

        plsc.subcore_barrier()
        obase = coff + r0
        pltpu.sync_copy(acc_sh.at[pl.ds(r0, RPT)], oacc_hbm.at[pl.ds(obase, RPT)])

        @pl.when(s == NS - 1)
        def _pub_tail():
            pltpu.sync_copy(acc_sh.at[pl.ds(NS * RPT, RTL)],
                            oacc_hbm.at[pl.ds(coff + NS * RPT, RTL)])

    return _attn(src, dst, q2, kv2)


def _combine_body(a0_ref, a1_ref, b0_ref, b1_ref, o_ref):
    num = jnp.concatenate([a0_ref[:, :VW], a1_ref[:, :VW]], axis=1)
    denb = (jnp.dot(a0_ref[:, DCOL:DCOL + HC], b0_ref[...],
                    preferred_element_type=jnp.float32)
            + jnp.dot(a1_ref[:, DCOL:DCOL + HC], b1_ref[...],
                      preferred_element_type=jnp.float32))
    o_ref[...] = num / (denb + 1e-16)


def _combine(acc, b0, b1):
    return pl.pallas_call(
        _combine_body,
        grid=(GRID,),
        in_specs=[
            pl.BlockSpec((RB, ACC_W), lambda i: (i, 0)),
            pl.BlockSpec((RB, ACC_W), lambda i: (i + GRID, 0)),
            pl.BlockSpec((HC, FV), lambda i: (0, 0)),
            pl.BlockSpec((HC, FV), lambda i: (0, 0)),
        ],
        out_specs=pl.BlockSpec((RB, FV), lambda i: (i, 0)),
        out_shape=jax.ShapeDtypeStruct((N, FV), jnp.float32),
    )(acc, acc, b0, b1)


_B0 = np.zeros((HC, FV), np.float32)
_B1 = np.zeros((HC, FV), np.float32)
for _h in range(HC):
    _B0[_h, _h * FH:(_h + 1) * FH] = 1.0
    _B1[_h, QW + _h * FH:QW + (_h + 1) * FH] = 1.0


def kernel(x, batch, ei, W):
    del batch
    src = ei[0].astype(jnp.int32)
    dst = ei[1].astype(jnp.int32)
    q2, kv2 = _project(x, W.T)
    acc = _sc_edges(src, dst,
                    q2.reshape(NC * N, QW), kv2.reshape(NC * N, KVW))
    return _combine(acc, jnp.asarray(_B0), jnp.asarray(_B1))


# speedup vs baseline: 1.5400x; 1.5400x over previous
import functools

import jax
import jax.numpy as jnp
import numpy as np
from jax import lax
from jax.experimental import pallas as pl
from jax.experimental.pallas import tpu as pltpu
from jax.experimental.pallas import tpu_sc as plsc

N = 10000
FIN = 128
FQK = 128
FV = 128
H = 8
FH = 16
E = 320000
SCALING = float(FH) ** -0.5

NC = 2
NS = 16
HC = H // NC
QW = HC * FH
VW = HC * FH
KVW = QW + VW
EPT = E // NS
CH = 80
NCHUNK = EPT // CH
RPT = 624
RTL = N - NS * RPT
ZCH = 48
ACC_W = 72
DCOL = VW

RB = 400
GRID = N // RB


def _proj_body(x_ref, wt_ref, q_ref, kv_ref):
    blk = jnp.dot(x_ref[...], wt_ref[...], preferred_element_type=jnp.float32)
    q_ref[0] = blk[:, :QW] * SCALING
    q_ref[1] = blk[:, QW:FQK] * SCALING
    kv_ref[0] = jnp.concatenate(
        [blk[:, FQK:FQK + QW], blk[:, 2 * FQK:2 * FQK + VW]], axis=1)
    kv_ref[1] = jnp.concatenate(
        [blk[:, FQK + QW:2 * FQK], blk[:, 2 * FQK + VW:]], axis=1)


def _project(x, wt):
    return pl.pallas_call(
        _proj_body,
        grid=(GRID,),
        in_specs=[
            pl.BlockSpec((RB, FIN), lambda i: (i, 0)),
            pl.BlockSpec((FIN, 2 * FQK + FV), lambda i: (0, 0)),
        ],
        out_specs=[
            pl.BlockSpec((NC, RB, QW), lambda i: (0, i, 0)),
            pl.BlockSpec((NC, RB, KVW), lambda i: (0, i, 0)),
        ],
        out_shape=[
            jax.ShapeDtypeStruct((NC, N, QW), jnp.float32),
            jax.ShapeDtypeStruct((NC, N, KVW), jnp.float32),
        ],
    )(x, wt)


def _sc_edges(src, dst, q2, kv2):
    mesh = plsc.VectorSubcoreMesh(core_axis_name="c", subcore_axis_name="s")

    @functools.partial(
        pl.kernel,
        mesh=mesh,
        compiler_params=pltpu.CompilerParams(
            needs_layout_passes=False, use_tc_tiling_on_sc=False),
        out_type=jax.ShapeDtypeStruct((NC * N, ACC_W), jnp.float32),
        scratch_types=[
            pltpu.VMEM((EPT,), jnp.int32),
            pltpu.VMEM((EPT,), jnp.int32),
            pltpu.VMEM((CH, QW), jnp.float32),
            pltpu.VMEM((CH, QW), jnp.float32),
            pltpu.VMEM((CH, KVW), jnp.float32),
            pltpu.VMEM((CH, KVW), jnp.float32),
            pltpu.VMEM((CH, ACC_W), jnp.float32),
            pltpu.VMEM((CH, ACC_W), jnp.float32),
            pltpu.VMEM((CH,), jnp.int32),
            pltpu.VMEM((CH,), jnp.int32),
            pltpu.VMEM_SHARED((N, ACC_W), jnp.float32),
            [pltpu.SemaphoreType.DMA] * 6,
        ],
    )
    def _attn(src_hbm, dst_hbm, q_hbm, kv_hbm, oacc_hbm,
              sadj, dadj, qr0, qr1, kvr0, kvr1, cb0, cb1,
              sc0, sc1, acc_sh, sems):
        c = lax.axis_index("c")
        s = lax.axis_index("s")
        coff = c * N
        ebase = s * EPT
        qr = (qr0, qr1)
        kvr = (kvr0, kvr1)
        cb = (cb0, cb1)
        sc = (sc0, sc1)
        semq = (sems[0], sems[1])
        semk = (sems[2], sems[3])
        sema = (sems[4], sems[5])

        zeros16 = jnp.zeros((FH,), jnp.float32)
        iota16 = lax.iota(jnp.int32, FH)
        cw = jnp.full((FH,), ACC_W, jnp.int32)

        cp0 = pltpu.async_copy(src_hbm.at[pl.ds(ebase, EPT)], sadj, semq[0])
        cp1 = pltpu.async_copy(dst_hbm.at[pl.ds(ebase, EPT)], dadj, semk[0])
        cp0.wait()
        cp1.wait()

        def _adj(j, _):
            sl = pl.ds(j * FH, FH)
            sadj[sl] = sadj[sl] + coff
            dadj[sl] = dadj[sl] + coff
            return 0

        lax.fori_loop(0, EPT // FH, _adj, 0)

        def _zero_flat(t, _):
            base = t * FH + iota16
            rows = lax.div(base, cw)
            cols = lax.rem(base, cw)
            plsc.store_scatter(cb0, [rows, cols], zeros16)
            plsc.store_scatter(cb1, [rows, cols], zeros16)
            return 0

        lax.fori_loop(0, CH * ACC_W // FH, _zero_flat, 0)
        r0 = s * RPT
        for z in range(RPT // ZCH):
            pltpu.sync_copy(cb0.at[pl.ds(0, ZCH)],
                            acc_sh.at[pl.ds(r0 + z * ZCH, ZCH)])

        @pl.when(s == NS - 1)
        def _zero_tail():
            pltpu.sync_copy(cb0.at[pl.ds(0, RTL)],
                            acc_sh.at[pl.ds(NS * RPT, RTL)])

        plsc.subcore_barrier()

        def _issue_gathers(off, b):
            pltpu.async_copy(q_hbm.at[sadj.at[pl.ds(off, CH)]], qr[b], semq[b])
            pltpu.async_copy(kv_hbm.at[dadj.at[pl.ds(off, CH)]], kvr[b], semk[b])

        idx_e = (2 * iota16) & 15
        idx_o = idx_e | 1
        mask_lo = iota16 < 8

        def _perm(a, idx):
            return jnp.take_along_axis(a, idx, axis=0,
                                       mode="promise_in_bounds")

        def _hadd(a, b):
            sa = _perm(a, idx_e) + _perm(a, idx_o)
            sb = _perm(b, idx_e) + _perm(b, idx_o)
            return jnp.where(mask_lo, sa, sb)

        def _compute(ngroup, qrb, kvrb, cbb):
            @plsc.parallel_loop(0, ngroup, 1, unroll=1)
            def _group(g):
                rows = g * FH + iota16
                for h in range(HC):
                    lvl = []
                    for j in range(FH):
                        e = g * FH + j
                        qv = qrb[e, pl.ds(h * FH, FH)]
                        kv = kvrb[e, pl.ds(h * FH, FH)]
                        lvl.append(qv * kv)
                    while len(lvl) > 1:
                        lvl = [_hadd(lvl[i], lvl[i + 1])
                               for i in range(0, len(lvl), 2)]
                    p = jnp.exp(lvl[0])
                    plsc.store_scatter(
                        cbb, [rows, jnp.full((FH,), DCOL + h, jnp.int32)], p)
                    for j in range(FH):
                        e = g * FH + j
                        pj = _perm(p, jnp.full((FH,), j, jnp.int32))
                        vv = kvrb[e, pl.ds(QW + h * FH, FH)]
                        cbb[e, pl.ds(h * FH, FH)] = pj * vv

        _issue_gathers(0, 0)
        _issue_gathers(CH, 1)

        def _pair(i2, _):
            for b in range(2):
                off = (i2 * 2 + b) * CH
                pltpu.make_async_copy(
                    q_hbm.at[sadj.at[pl.ds(off, CH)]], qr[b], semq[b]).wait()
                pltpu.make_async_copy(
                    kv_hbm.at[dadj.at[pl.ds(off, CH)]], kvr[b], semk[b]).wait()

                @pl.when(i2 >= 1)
                def _wait_scatter():
                    pltpu.make_async_copy(
                        cb[b], acc_sh.at[sc[b]], sema[b]).wait()

                for j in range(CH // FH):
                    sc[b][pl.ds(j * FH, FH)] = (
                        sadj[pl.ds(off + j * FH, FH)] - coff)
                _compute(CH // FH, qr[b], kvr[b], cb[b])
                pltpu.async_copy(cb[b], acc_sh.at[sc[b]], sema[b], add=True)

                @pl.when(i2 < NCHUNK // 2 - 1)
                def _issue_next():
                    _issue_gathers(off + 2 * CH, b)
            return 0

        lax.fori_loop(0, NCHUNK // 2, _pair, 0)
        for b in range(2):
            pltpu.make_async_copy(cb[b], acc_sh.at[sc[b]], sema[b]).wait()

        plsc.subcore_barrier()
        obase = coff + r0
        pltpu.sync_copy(acc_sh.at[pl.ds(r0, RPT)], oacc_hbm.at[pl.ds(obase, RPT)])

        @pl.when(s == NS - 1)
        def _pub_tail():
            pltpu.sync_copy(acc_sh.at[pl.ds(NS * RPT, RTL)],
                            oacc_hbm.at[pl.ds(coff + NS * RPT, RTL)])

    return _attn(src, dst, q2, kv2)


def _combine_body(a0_ref, a1_ref, b0_ref, b1_ref, o_ref):
    num = jnp.concatenate([a0_ref[:, :VW], a1_ref[:, :VW]], axis=1)
    denb = (jnp.dot(a0_ref[:, DCOL:DCOL + HC], b0_ref[...],
                    preferred_element_type=jnp.float32)
            + jnp.dot(a1_ref[:, DCOL:DCOL + HC], b1_ref[...],
                      preferred_element_type=jnp.float32))
    o_ref[...] = num / (denb + 1e-16)


def _combine(acc, b0, b1):
    return pl.pallas_call(
        _combine_body,
        grid=(GRID,),
        in_specs=[
            pl.BlockSpec((RB, ACC_W), lambda i: (i, 0)),
            pl.BlockSpec((RB, ACC_W), lambda i: (i + GRID, 0)),
            pl.BlockSpec((HC, FV), lambda i: (0, 0)),
            pl.BlockSpec((HC, FV), lambda i: (0, 0)),
        ],
        out_specs=pl.BlockSpec((RB, FV), lambda i: (i, 0)),
        out_shape=jax.ShapeDtypeStruct((N, FV), jnp.float32),
    )(acc, acc, b0, b1)


_B0 = np.zeros((HC, FV), np.float32)
_B1 = np.zeros((HC, FV), np.float32)
for _h in range(HC):
    _B0[_h, _h * FH:(_h + 1) * FH] = 1.0
    _B1[_h, QW + _h * FH:QW + (_h + 1) * FH] = 1.0


def kernel(x, batch, ei, W):
    del batch
    src = ei[0].astype(jnp.int32)
    dst = ei[1].astype(jnp.int32)
    q2, kv2 = _project(x, W.T)
    acc = _sc_edges(src, dst,
                    q2.reshape(NC * N, QW), kv2.reshape(NC * N, KVW))
    return _combine(acc, jnp.asarray(_B0), jnp.asarray(_B1))


# speedup vs baseline: 1.6246x; 1.0549x over previous
import functools

import jax
import jax.numpy as jnp
import numpy as np
from jax import lax
from jax.experimental import pallas as pl
from jax.experimental.pallas import tpu as pltpu
from jax.experimental.pallas import tpu_sc as plsc

N = 10000
FIN = 128
FQK = 128
FV = 128
H = 8
FH = 16
E = 320000
SCALING = float(FH) ** -0.5

NC = 2
NS = 16
HC = H // NC
QW = HC * FH
VW = HC * FH
KVW = QW + VW
EPT = E // NS
CH = 80
NCHUNK = EPT // CH
RPT = 624
RTL = N - NS * RPT
ZCH = 48
ACC_W = 72
DCOL = VW

RB = 400
GRID = N // RB


def _proj_body(x_ref, wt_ref, q_ref, kv_ref):
    blk = jnp.dot(x_ref[...], wt_ref[...], preferred_element_type=jnp.float32)
    q_ref[0] = blk[:, :QW] * SCALING
    q_ref[1] = blk[:, QW:FQK] * SCALING
    kv_ref[0] = jnp.concatenate(
        [blk[:, FQK:FQK + QW], blk[:, 2 * FQK:2 * FQK + VW]], axis=1)
    kv_ref[1] = jnp.concatenate(
        [blk[:, FQK + QW:2 * FQK], blk[:, 2 * FQK + VW:]], axis=1)


def _project(x, wt):
    return pl.pallas_call(
        _proj_body,
        grid=(GRID,),
        in_specs=[
            pl.BlockSpec((RB, FIN), lambda i: (i, 0)),
            pl.BlockSpec((FIN, 2 * FQK + FV), lambda i: (0, 0)),
        ],
        out_specs=[
            pl.BlockSpec((NC, RB, QW), lambda i: (0, i, 0)),
            pl.BlockSpec((NC, RB, KVW), lambda i: (0, i, 0)),
        ],
        out_shape=[
            jax.ShapeDtypeStruct((NC, N, QW), jnp.float32),
            jax.ShapeDtypeStruct((NC, N, KVW), jnp.float32),
        ],
    )(x, wt)


def _sc_edges(src, dst, q2, kv2):
    mesh = plsc.VectorSubcoreMesh(core_axis_name="c", subcore_axis_name="s")

    @functools.partial(
        pl.kernel,
        mesh=mesh,
        compiler_params=pltpu.CompilerParams(
            needs_layout_passes=False, use_tc_tiling_on_sc=False),
        out_type=jax.ShapeDtypeStruct((NC * N, ACC_W), jnp.float32),
        scratch_types=[
            pltpu.VMEM((EPT,), jnp.int32),
            pltpu.VMEM((EPT,), jnp.int32),
            pltpu.VMEM((CH, QW), jnp.float32),
            pltpu.VMEM((CH, QW), jnp.float32),
            pltpu.VMEM((CH, KVW), jnp.float32),
            pltpu.VMEM((CH, KVW), jnp.float32),
            pltpu.VMEM((CH, ACC_W), jnp.float32),
            pltpu.VMEM((CH, ACC_W), jnp.float32),
            pltpu.VMEM((CH,), jnp.int32),
            pltpu.VMEM((CH,), jnp.int32),
            pltpu.VMEM_SHARED((N, ACC_W), jnp.float32),
            [pltpu.SemaphoreType.DMA] * 6,
        ],
    )
    def _attn(src_hbm, dst_hbm, q_hbm, kv_hbm, oacc_hbm,
              sadj, dadj, qr0, qr1, kvr0, kvr1, cb0, cb1,
              sc0, sc1, acc_sh, sems):
        c = lax.axis_index("c")
        s = lax.axis_index("s")
        coff = c * N
        ebase = s * EPT
        qr = (qr0, qr1)
        kvr = (kvr0, kvr1)
        cb = (cb0, cb1)
        sc = (sc0, sc1)
        semq = (sems[0], sems[1])
        semk = (sems[2], sems[3])
        sema = (sems[4], sems[5])

        zeros16 = jnp.zeros((FH,), jnp.float32)
        iota16 = lax.iota(jnp.int32, FH)
        cw = jnp.full((FH,), ACC_W, jnp.int32)

        cp0 = pltpu.async_copy(src_hbm.at[pl.ds(ebase, EPT)], sadj, semq[0])
        cp1 = pltpu.async_copy(dst_hbm.at[pl.ds(ebase, EPT)], dadj, semk[0])
        cp0.wait()
        cp1.wait()

        def _adj(j, _):
            sl = pl.ds(j * FH, FH)
            sadj[sl] = sadj[sl] + coff
            dadj[sl] = dadj[sl] + coff
            return 0

        lax.fori_loop(0, EPT // FH, _adj, 0)

        def _zero_flat(t, _):
            base = t * FH + iota16
            rows = lax.div(base, cw)
            cols = lax.rem(base, cw)
            plsc.store_scatter(cb0, [rows, cols], zeros16)
            plsc.store_scatter(cb1, [rows, cols], zeros16)
            return 0

        lax.fori_loop(0, CH * ACC_W // FH, _zero_flat, 0)
        r0 = s * RPT
        for z in range(RPT // ZCH):
            pltpu.sync_copy(cb0.at[pl.ds(0, ZCH)],
                            acc_sh.at[pl.ds(r0 + z * ZCH, ZCH)])

        @pl.when(s == NS - 1)
        def _zero_tail():
            pltpu.sync_copy(cb0.at[pl.ds(0, RTL)],
                            acc_sh.at[pl.ds(NS * RPT, RTL)])

        plsc.subcore_barrier()

        def _issue_gathers(off, b):
            pltpu.async_copy(q_hbm.at[sadj.at[pl.ds(off, CH)]], qr[b], semq[b])
            pltpu.async_copy(kv_hbm.at[dadj.at[pl.ds(off, CH)]], kvr[b], semk[b])

        idx_e = (2 * iota16) & 15
        idx_o = idx_e | 1
        mask_lo = iota16 < 8

        def _perm(a, idx):
            return jnp.take_along_axis(a, idx, axis=0,
                                       mode="promise_in_bounds")

        def _hadd(a, b):
            sa = _perm(a, idx_e) + _perm(a, idx_o)
            sb = _perm(b, idx_e) + _perm(b, idx_o)
            return jnp.where(mask_lo, sa, sb)

        def _compute(ngroup, qrb, kvrb, cbb):
            @plsc.parallel_loop(0, ngroup, 1, unroll=1)
            def _group(g):
                rows = g * FH + iota16
                for hp in range(HC // 2):
                    hs = (2 * hp, 2 * hp + 1)
                    lvls = [[], []]
                    for j in range(FH):
                        e = g * FH + j
                        for k, h in enumerate(hs):
                            qv = qrb[e, pl.ds(h * FH, FH)]
                            kv = kvrb[e, pl.ds(h * FH, FH)]
                            lvls[k].append(qv * kv)
                    while len(lvls[0]) > 1:
                        lvls = [[_hadd(L[i], L[i + 1])
                                 for i in range(0, len(L), 2)] for L in lvls]
                    ps = [jnp.exp(L[0]) for L in lvls]
                    for k, h in enumerate(hs):
                        plsc.store_scatter(
                            cbb, [rows, jnp.full((FH,), DCOL + h, jnp.int32)],
                            ps[k])
                    for j in range(FH):
                        e = g * FH + j
                        for k, h in enumerate(hs):
                            pj = _perm(ps[k], jnp.full((FH,), j, jnp.int32))
                            vv = kvrb[e, pl.ds(QW + h * FH, FH)]
                            cbb[e, pl.ds(h * FH, FH)] = pj * vv

        _issue_gathers(0, 0)
        _issue_gathers(CH, 1)

        def _pair(i2, _):
            for b in range(2):
                off = (i2 * 2 + b) * CH
                pltpu.make_async_copy(
                    q_hbm.at[sadj.at[pl.ds(off, CH)]], qr[b], semq[b]).wait()
                pltpu.make_async_copy(
                    kv_hbm.at[dadj.at[pl.ds(off, CH)]], kvr[b], semk[b]).wait()

                @pl.when(i2 >= 1)
                def _wait_scatter():
                    pltpu.make_async_copy(
                        cb[b], acc_sh.at[sc[b]], sema[b]).wait()

                for j in range(CH // FH):
                    sc[b][pl.ds(j * FH, FH)] = (
                        sadj[pl.ds(off + j * FH, FH)] - coff)
                _compute(CH // FH, qr[b], kvr[b], cb[b])
                pltpu.async_copy(cb[b], acc_sh.at[sc[b]], sema[b], add=True)

                @pl.when(i2 < NCHUNK // 2 - 1)
                def _issue_next():
                    _issue_gathers(off + 2 * CH, b)
            return 0

        lax.fori_loop(0, NCHUNK // 2, _pair, 0)
        for b in range(2):
            pltpu.make_async_copy(cb[b], acc_sh.at[sc[b]], sema[b]).wait()

        plsc.subcore_barrier()
        obase = coff + r0
        pltpu.sync_copy(acc_sh.at[pl.ds(r0, RPT)], oacc_hbm.at[pl.ds(obase, RPT)])

        @pl.when(s == NS - 1)
        def _pub_tail():
            pltpu.sync_copy(acc_sh.at[pl.ds(NS * RPT, RTL)],
                            oacc_hbm.at[pl.ds(coff + NS * RPT, RTL)])

    return _attn(src, dst, q2, kv2)


def _combine_body(a0_ref, a1_ref, b0_ref, b1_ref, o_ref):
    num = jnp.concatenate([a0_ref[:, :VW], a1_ref[:, :VW]], axis=1)
    denb = (jnp.dot(a0_ref[:, DCOL:DCOL + HC], b0_ref[...],
                    preferred_element_type=jnp.float32)
            + jnp.dot(a1_ref[:, DCOL:DCOL + HC], b1_ref[...],
                      preferred_element_type=jnp.float32))
    o_ref[...] = num / (denb + 1e-16)


def _combine(acc, b0, b1):
    return pl.pallas_call(
        _combine_body,
        grid=(GRID,),
        in_specs=[
            pl.BlockSpec((RB, ACC_W), lambda i: (i, 0)),
            pl.BlockSpec((RB, ACC_W), lambda i: (i + GRID, 0)),
            pl.BlockSpec((HC, FV), lambda i: (0, 0)),
            pl.BlockSpec((HC, FV), lambda i: (0, 0)),
        ],
        out_specs=pl.BlockSpec((RB, FV), lambda i: (i, 0)),
        out_shape=jax.ShapeDtypeStruct((N, FV), jnp.float32),
    )(acc, acc, b0, b1)


_B0 = np.zeros((HC, FV), np.float32)
_B1 = np.zeros((HC, FV), np.float32)
for _h in range(HC):
    _B0[_h, _h * FH:(_h + 1) * FH] = 1.0
    _B1[_h, QW + _h * FH:QW + (_h + 1) * FH] = 1.0


def kernel(x, batch, ei, W):
    del batch
    src = ei[0].astype(jnp.int32)
    dst = ei[1].astype(jnp.int32)
    q2, kv2 = _project(x, W.T)
    acc = _sc_edges(src, dst,
                    q2.reshape(NC * N, QW), kv2.reshape(NC * N, KVW))
    return _combine(acc, jnp.asarray(_B0), jnp.asarray(_B1))
